# baseline (device time: 255234 ns/iter reference)
import jax
import jax.numpy as jnp
from jax import lax
from jax.experimental import pallas as pl
from jax.experimental.pallas import tpu as pltpu

CHUNK_ROWS = [64, 64, 128, 256] + [512] * 7
K = len(CHUNK_ROWS)
CHUNK_OFF = [sum(CHUNK_ROWS[:i]) for i in range(K)]

KL = 8
SLOTS = 4
LOCAL_AT = 3


def kernel(x):
    m, n = x.shape
    half = m // 2
    chl = m // KL

    def body(x_ref, out_ref, stage, send_x, recv_x, send_y, recv_y,
             load_sems, store_sems):
        my_x = lax.axis_index("x")
        my_y = lax.axis_index("y")
        x_peer = (1 - my_x, my_y)
        y_peer = (my_x, 1 - my_y)

        barrier = pltpu.get_barrier_semaphore()
        for nbr in (x_peer, y_peer):
            pl.semaphore_signal(
                barrier, inc=1, device_id=nbr,
                device_id_type=pl.DeviceIdType.MESH,
            )
        pl.semaphore_wait(barrier, 2)

        loads = [None] * KL
        stores = [None] * KL

        def start_load(c):
            ld = pltpu.make_async_copy(
                x_ref.at[pl.ds(c * chl, chl)], stage.at[c % SLOTS],
                load_sems.at[c],
            )
            ld.start()
            loads[c] = ld

        def local_step(c):
            if c >= 1 and c + 3 < KL:
                stores[c - 1].wait()
                start_load(c + 3)
            loads[c].wait()
            st = pltpu.make_async_copy(
                stage.at[c % SLOTS],
                out_ref.at[pl.ds(my_x * m + c * chl, chl)],
                store_sems.at[c],
            )
            st.start()
            stores[c] = st

        for c in range(SLOTS):
            start_load(c)

        send_base = my_y * half
        dst_base = my_x * m + my_y * half
        xr = []
        for c in range(K):
            r = pltpu.make_async_remote_copy(
                src_ref=x_ref.at[pl.ds(send_base + CHUNK_OFF[c],
                                       CHUNK_ROWS[c])],
                dst_ref=out_ref.at[pl.ds(dst_base + CHUNK_OFF[c],
                                         CHUNK_ROWS[c])],
                send_sem=send_x.at[c],
                recv_sem=recv_x.at[c],
                device_id=x_peer,
                device_id_type=pl.DeviceIdType.MESH,
            )
            r.start()
            xr.append(r)

        fwd_base = (1 - my_x) * m + my_y * half
        yr = []
        for c in range(K):
            xr[c].wait_recv()
            r = pltpu.make_async_remote_copy(
                src_ref=out_ref.at[pl.ds(fwd_base + CHUNK_OFF[c],
                                         CHUNK_ROWS[c])],
                dst_ref=out_ref.at[pl.ds(fwd_base + CHUNK_OFF[c],
                                         CHUNK_ROWS[c])],
                send_sem=send_y.at[c],
                recv_sem=recv_y.at[c],
                device_id=y_peer,
                device_id_type=pl.DeviceIdType.MESH,
            )
            r.start()
            yr.append(r)
            if LOCAL_AT <= c < LOCAL_AT + KL:
                local_step(c - LOCAL_AT)

        for c in range(K):
            xr[c].wait_send()
            yr[c].wait()
        for c in range(KL - SLOTS, KL):
            stores[c].wait()

    return pl.pallas_call(
        body,
        out_shape=jax.ShapeDtypeStruct((2 * m, n), x.dtype),
        in_specs=[pl.BlockSpec(memory_space=pl.ANY)],
        out_specs=pl.BlockSpec(memory_space=pl.ANY),
        scratch_shapes=[
            pltpu.VMEM((SLOTS, chl, n), x.dtype),
            pltpu.SemaphoreType.DMA((K,)),
            pltpu.SemaphoreType.DMA((K,)),
            pltpu.SemaphoreType.DMA((K,)),
            pltpu.SemaphoreType.DMA((K,)),
            pltpu.SemaphoreType.DMA((KL,)),
            pltpu.SemaphoreType.DMA((KL,)),
        ],
        compiler_params=pltpu.CompilerParams(collective_id=0),
    )(x)


# device time: 239264 ns/iter; 1.0667x vs baseline; 1.0667x over previous
import jax
import jax.numpy as jnp
from jax import lax
from jax.experimental import pallas as pl
from jax.experimental.pallas import tpu as pltpu

CHUNK_ROWS = [128] * 32
K = len(CHUNK_ROWS)
CHUNK_OFF = [sum(CHUNK_ROWS[:i]) for i in range(K)]

KL = 8
SLOTS = 4
LOCAL_AT = 4
LOCAL_STRIDE = 3


def kernel(x):
    m, n = x.shape
    half = m // 2
    chl = m // KL

    def body(x_ref, out_ref, stage, send_x, recv_x, send_y, recv_y,
             load_sems, store_sems):
        my_x = lax.axis_index("x")
        my_y = lax.axis_index("y")
        x_peer = (1 - my_x, my_y)
        y_peer = (my_x, 1 - my_y)

        barrier = pltpu.get_barrier_semaphore()
        for nbr in (x_peer, y_peer):
            pl.semaphore_signal(
                barrier, inc=1, device_id=nbr,
                device_id_type=pl.DeviceIdType.MESH,
            )
        pl.semaphore_wait(barrier, 2)

        loads = [None] * KL
        stores = [None] * KL

        def start_load(c):
            ld = pltpu.make_async_copy(
                x_ref.at[pl.ds(c * chl, chl)], stage.at[c % SLOTS],
                load_sems.at[c],
            )
            ld.start()
            loads[c] = ld

        def local_step(c):
            if c >= 1 and c + 3 < KL:
                stores[c - 1].wait()
                start_load(c + 3)
            loads[c].wait()
            st = pltpu.make_async_copy(
                stage.at[c % SLOTS],
                out_ref.at[pl.ds(my_x * m + c * chl, chl)],
                store_sems.at[c],
            )
            st.start()
            stores[c] = st

        for c in range(SLOTS):
            start_load(c)

        send_base = my_y * half
        dst_base = my_x * m + my_y * half
        xr = []
        for c in range(K):
            r = pltpu.make_async_remote_copy(
                src_ref=x_ref.at[pl.ds(send_base + CHUNK_OFF[c],
                                       CHUNK_ROWS[c])],
                dst_ref=out_ref.at[pl.ds(dst_base + CHUNK_OFF[c],
                                         CHUNK_ROWS[c])],
                send_sem=send_x.at[c],
                recv_sem=recv_x.at[c],
                device_id=x_peer,
                device_id_type=pl.DeviceIdType.MESH,
            )
            r.start()
            xr.append(r)

        fwd_base = (1 - my_x) * m + my_y * half
        yr = []
        for c in range(K):
            xr[c].wait_recv()
            r = pltpu.make_async_remote_copy(
                src_ref=out_ref.at[pl.ds(fwd_base + CHUNK_OFF[c],
                                         CHUNK_ROWS[c])],
                dst_ref=out_ref.at[pl.ds(fwd_base + CHUNK_OFF[c],
                                         CHUNK_ROWS[c])],
                send_sem=send_y.at[c],
                recv_sem=recv_y.at[c],
                device_id=y_peer,
                device_id_type=pl.DeviceIdType.MESH,
            )
            r.start()
            yr.append(r)
            if c >= LOCAL_AT and (c - LOCAL_AT) % LOCAL_STRIDE == 0:
                step = (c - LOCAL_AT) // LOCAL_STRIDE
                if step < KL:
                    local_step(step)

        for c in range(K):
            xr[c].wait_send()
            yr[c].wait()
        for c in range(KL - SLOTS, KL):
            stores[c].wait()

    return pl.pallas_call(
        body,
        out_shape=jax.ShapeDtypeStruct((2 * m, n), x.dtype),
        in_specs=[pl.BlockSpec(memory_space=pl.ANY)],
        out_specs=pl.BlockSpec(memory_space=pl.ANY),
        scratch_shapes=[
            pltpu.VMEM((SLOTS, chl, n), x.dtype),
            pltpu.SemaphoreType.DMA((K,)),
            pltpu.SemaphoreType.DMA((K,)),
            pltpu.SemaphoreType.DMA((K,)),
            pltpu.SemaphoreType.DMA((K,)),
            pltpu.SemaphoreType.DMA((KL,)),
            pltpu.SemaphoreType.DMA((KL,)),
        ],
        compiler_params=pltpu.CompilerParams(collective_id=0),
    )(x)


# device time: 237470 ns/iter; 1.0748x vs baseline; 1.0076x over previous
import jax
import jax.numpy as jnp
from jax import lax
from jax.experimental import pallas as pl
from jax.experimental.pallas import tpu as pltpu

CHUNK_ROWS = [64] * 64
K = len(CHUNK_ROWS)
CHUNK_OFF = [sum(CHUNK_ROWS[:i]) for i in range(K)]

KL = 8
SLOTS = 4
LOCAL_AT = 4
LOCAL_STRIDE = 7


def kernel(x):
    m, n = x.shape
    half = m // 2
    chl = m // KL

    def body(x_ref, out_ref, stage, send_x, recv_x, send_y, recv_y,
             load_sems, store_sems):
        my_x = lax.axis_index("x")
        my_y = lax.axis_index("y")
        x_peer = (1 - my_x, my_y)
        y_peer = (my_x, 1 - my_y)

        barrier = pltpu.get_barrier_semaphore()
        for nbr in (x_peer, y_peer):
            pl.semaphore_signal(
                barrier, inc=1, device_id=nbr,
                device_id_type=pl.DeviceIdType.MESH,
            )
        pl.semaphore_wait(barrier, 2)

        loads = [None] * KL
        stores = [None] * KL

        def start_load(c):
            ld = pltpu.make_async_copy(
                x_ref.at[pl.ds(c * chl, chl)], stage.at[c % SLOTS],
                load_sems.at[c],
            )
            ld.start()
            loads[c] = ld

        def local_step(c):
            if c >= 1 and c + 3 < KL:
                stores[c - 1].wait()
                start_load(c + 3)
            loads[c].wait()
            st = pltpu.make_async_copy(
                stage.at[c % SLOTS],
                out_ref.at[pl.ds(my_x * m + c * chl, chl)],
                store_sems.at[c],
            )
            st.start()
            stores[c] = st

        for c in range(SLOTS):
            start_load(c)

        send_base = my_y * half
        dst_base = my_x * m + my_y * half
        xr = []
        for c in range(K):
            r = pltpu.make_async_remote_copy(
                src_ref=x_ref.at[pl.ds(send_base + CHUNK_OFF[c],
                                       CHUNK_ROWS[c])],
                dst_ref=out_ref.at[pl.ds(dst_base + CHUNK_OFF[c],
                                         CHUNK_ROWS[c])],
                send_sem=send_x.at[c],
                recv_sem=recv_x.at[c],
                device_id=x_peer,
                device_id_type=pl.DeviceIdType.MESH,
            )
            r.start()
            xr.append(r)

        fwd_base = (1 - my_x) * m + my_y * half
        yr = []
        for c in range(K):
            xr[c].wait_recv()
            r = pltpu.make_async_remote_copy(
                src_ref=out_ref.at[pl.ds(fwd_base + CHUNK_OFF[c],
                                         CHUNK_ROWS[c])],
                dst_ref=out_ref.at[pl.ds(fwd_base + CHUNK_OFF[c],
                                         CHUNK_ROWS[c])],
                send_sem=send_y.at[c],
                recv_sem=recv_y.at[c],
                device_id=y_peer,
                device_id_type=pl.DeviceIdType.MESH,
            )
            r.start()
            yr.append(r)
            if c >= LOCAL_AT and (c - LOCAL_AT) % LOCAL_STRIDE == 0:
                step = (c - LOCAL_AT) // LOCAL_STRIDE
                if step < KL:
                    local_step(step)

        for c in range(K):
            xr[c].wait_send()
            yr[c].wait()
        for c in range(KL - SLOTS, KL):
            stores[c].wait()

    return pl.pallas_call(
        body,
        out_shape=jax.ShapeDtypeStruct((2 * m, n), x.dtype),
        in_specs=[pl.BlockSpec(memory_space=pl.ANY)],
        out_specs=pl.BlockSpec(memory_space=pl.ANY),
        scratch_shapes=[
            pltpu.VMEM((SLOTS, chl, n), x.dtype),
            pltpu.SemaphoreType.DMA((K,)),
            pltpu.SemaphoreType.DMA((K,)),
            pltpu.SemaphoreType.DMA((K,)),
            pltpu.SemaphoreType.DMA((K,)),
            pltpu.SemaphoreType.DMA((KL,)),
            pltpu.SemaphoreType.DMA((KL,)),
        ],
        compiler_params=pltpu.CompilerParams(collective_id=0),
    )(x)
